# baseline (speedup 1.0000x reference)
import functools

import jax
import jax.numpy as jnp
from jax import lax
from jax.experimental import pallas as pl
from jax.experimental.pallas import tpu as pltpu
from jax.experimental.pallas import tpu_sc as plsc

N = 10000
E = 320000
NC, NS, L = 2, 16, 16
NW = NC * NS
EROW = 128
CE = 10240
EP = NW * CE
ACC = 10240
LROWS = ACC // NW
TRASH_L = 316
REG = 14336
RB = 1000
GRID = N // RB


def _mesh():
    return plsc.VectorSubcoreMesh(core_axis_name="c", subcore_axis_name="s",
                                  num_cores=NC, num_subcores=NS)


def _sc_params():
    return pltpu.CompilerParams(use_tc_tiling_on_sc=False,
                                needs_layout_passes=False)


def _fill1(buf, n, value, dtype):
    v = jnp.full((L,), value, dtype)
    def body(i, _):
        buf[pl.ds(pl.multiple_of(i * L, L), L)] = v
        return 0
    lax.fori_loop(0, n // L, body, 0)


def _fill2(buf, rows, cols, value, dtype):
    v = jnp.full((L,), value, dtype)
    def body(i, _):
        r = i // (cols // L)
        j = i % (cols // L)
        buf[r, pl.ds(pl.multiple_of(j * L, L), L)] = v
        return 0
    lax.fori_loop(0, rows * (cols // L), body, 0)


def _splat(x):
    return jnp.full((L,), x, jnp.int32)


@functools.lru_cache(maxsize=None)
def _make_reorder():
    @functools.partial(
        pl.kernel,
        out_type=(jax.ShapeDtypeStruct((NW * REG,), jnp.int32),
                  jax.ShapeDtypeStruct((NW * REG,), jnp.int32),
                  jax.ShapeDtypeStruct((NW * 64,), jnp.int32)),
        mesh=_mesh(),
        compiler_params=_sc_params(),
        scratch_types=[
            pltpu.VMEM((CE,), jnp.int32),
            pltpu.VMEM((CE,), jnp.int32),
            pltpu.VMEM((REG,), jnp.int32),
            pltpu.VMEM((REG,), jnp.int32),
            pltpu.VMEM((64,), jnp.int32),
            pltpu.SMEM((32,), jnp.int32),
        ],
    )
    def reorder(srcf, dstf, stage_s, stage_d, tab, sidx, didx, ss, sd,
                tab_row, cur):
        c = lax.axis_index("c")
        s = lax.axis_index("s")
        w = s * NC + c

        pltpu.sync_copy(srcf.at[pl.ds(pl.multiple_of(w * CE, 8), CE)], sidx)
        pltpu.sync_copy(dstf.at[pl.ds(pl.multiple_of(w * CE, 8), CE)], didx)

        _fill1(ss, REG, 0, jnp.int32)
        _fill1(sd, REG, TRASH_L, jnp.int32)

        for b in range(32):
            cur[b] = 0
        def count_chunk(t, _):
            dv = didx[pl.ds(pl.multiple_of(t * L, L), L)]
            bv = dv & 31
            for l in range(L):
                b = bv[l]
                cur[b] = cur[b] + 1
            return 0
        lax.fori_loop(0, CE // L, count_chunk, 0)

        off = 0
        for b in range(32):
            cnt = cur[b]
            cntp = ((cnt + EROW - 1) // EROW) * EROW
            plsc.store_scatter(tab_row, [_splat(b)], _splat(off))
            plsc.store_scatter(tab_row, [_splat(b + 32)], _splat(cntp))
            cur[b] = off
            off = off + cntp

        def place_chunk(t, _):
            to = pl.multiple_of(t * L, L)
            sv = sidx[pl.ds(to, L)]
            dv = didx[pl.ds(to, L)]
            bv = dv & 31
            qv = dv >> 5
            for l in range(L):
                b = bv[l]
                pos = cur[b]
                plsc.store_scatter(ss, [_splat(pos)], _splat(sv[l]))
                plsc.store_scatter(sd, [_splat(pos)], _splat(qv[l]))
                cur[b] = pos + 1
            return 0
        lax.fori_loop(0, CE // L, place_chunk, 0)

        pltpu.sync_copy(ss, stage_s.at[pl.ds(pl.multiple_of(w * REG, 8), REG)])
        pltpu.sync_copy(sd, stage_d.at[pl.ds(pl.multiple_of(w * REG, 8), REG)])
        pltpu.sync_copy(tab_row, tab.at[pl.ds(pl.multiple_of(w * 64, 8), 64)])

    return reorder


def _seg_entries(tab_v, w, B):
    off = plsc.load_gather(tab_v, [_splat(w * 64 + B)])[0]
    cntp = plsc.load_gather(tab_v, [_splat(w * 64 + 32 + B)])[0]
    return off, cntp


@functools.lru_cache(maxsize=None)
def _make_pass(F):
    @functools.partial(
        pl.kernel,
        out_type=jax.ShapeDtypeStruct((NW, LROWS, F), jnp.float32),
        mesh=_mesh(),
        compiler_params=_sc_params(),
        scratch_types=[
            pltpu.VMEM((NW * 64,), jnp.int32),
            pltpu.VMEM((EROW,), jnp.int32),
            pltpu.VMEM((EROW,), jnp.int32),
            pltpu.VMEM((EROW, F), jnp.float32),
            pltpu.VMEM((LROWS, F), jnp.float32),
            pltpu.SemaphoreType.DMA,
        ],
    )
    def prop(g_hbm, stage_s, stage_d, tab, out_hbm,
             tab_v, sbuf, dbuf, bufG, acc, sem):
        c = lax.axis_index("c")
        s = lax.axis_index("s")
        B = s * NC + c

        _fill2(acc, LROWS, F, 0.0, jnp.float32)
        pltpu.sync_copy(tab, tab_v)
        iota16 = lax.iota(jnp.int32, 16)

        def wloop(w, _):
            off, cntp = _seg_entries(tab_v, w, B)
            def kloop(k, _):
                base = pl.multiple_of(w * REG + off + k * EROW, 8)
                pltpu.sync_copy(stage_s.at[pl.ds(base, EROW)], sbuf)
                pltpu.sync_copy(stage_d.at[pl.ds(base, EROW)], dbuf)
                pltpu.async_copy(g_hbm.at[sbuf], bufG, sem).wait()
                def chunk16(t, _):
                    to = pl.multiple_of(t * L, L)
                    dl16 = dbuf[pl.ds(to, L)]
                    e16 = iota16 + t * L
                    for j in range(F):
                        col = jnp.full((L,), j, jnp.int32)
                        vals = plsc.load_gather(bufG, [e16, col])
                        plsc.addupdate_scatter(acc, [dl16, col], vals)
                    return 0
                lax.fori_loop(0, EROW // L, chunk16, 0)
                return 0
            lax.fori_loop(0, cntp // EROW, kloop, 0)
            return 0
        lax.fori_loop(0, NW, wloop, 0)

        pltpu.sync_copy(acc, out_hbm.at[B])

    return prop


@functools.lru_cache(maxsize=None)
def _make_deg_pass():
    @functools.partial(
        pl.kernel,
        out_type=jax.ShapeDtypeStruct((NW, LROWS), jnp.float32),
        mesh=_mesh(),
        compiler_params=_sc_params(),
        scratch_types=[
            pltpu.VMEM((NW * 64,), jnp.int32),
            pltpu.VMEM((EROW,), jnp.int32),
            pltpu.VMEM((LROWS, L), jnp.float32),
            pltpu.VMEM((LROWS,), jnp.float32),
        ],
    )
    def degp(stage_d, tab, out_hbm, tab_v, dbuf, acc, orow):
        c = lax.axis_index("c")
        s = lax.axis_index("s")
        B = s * NC + c

        _fill2(acc, LROWS, L, 0.0, jnp.float32)
        pltpu.sync_copy(tab, tab_v)
        zero16 = jnp.zeros((L,), jnp.int32)
        one16f = jnp.ones((L,), jnp.float32)

        def wloop(w, _):
            off, cntp = _seg_entries(tab_v, w, B)
            def kloop(k, _):
                base = pl.multiple_of(w * REG + off + k * EROW, 8)
                pltpu.sync_copy(stage_d.at[pl.ds(base, EROW)], dbuf)
                def chunk16(t, _):
                    to = pl.multiple_of(t * L, L)
                    dl16 = dbuf[pl.ds(to, L)]
                    plsc.addupdate_scatter(acc, [dl16, zero16], one16f)
                    return 0
                lax.fori_loop(0, EROW // L, chunk16, 0)
                return 0
            lax.fori_loop(0, cntp // EROW, kloop, 0)
            return 0
        lax.fori_loop(0, NW, wloop, 0)

        iota16 = lax.iota(jnp.int32, 16)
        def collect(i, _):
            rows = iota16 + i * L
            orow[pl.ds(pl.multiple_of(i * L, L), L)] = plsc.load_gather(
                acc, [rows, jnp.zeros((L,), jnp.int32)])
            return 0
        lax.fori_loop(0, LROWS // L, collect, 0)
        pltpu.sync_copy(orow, out_hbm.at[B])

    return degp


def _prop(F, g, stage_s, stage_d, tab):
    out = _make_pass(F)(g, stage_s, stage_d, tab)
    return out.transpose(1, 0, 2).reshape(ACC, F)


def _row(F):
    return pl.BlockSpec((RB, F), lambda i: (i, 0))


def _full(shape):
    return pl.BlockSpec(shape, lambda i: tuple(0 for _ in shape))


def _tc(body, in_specs, out_specs, out_shape):
    if not isinstance(out_shape, (tuple, list)):
        out_specs = out_specs[0]
    return pl.pallas_call(body, grid=(GRID,), in_specs=in_specs,
                          out_specs=out_specs, out_shape=out_shape)


def _k0_body(sdeg, x, dinv_o, g0_o):
    deg = sdeg[...] + 1.0
    di = lax.rsqrt(deg)
    dinv_o[...] = di
    g0_o[...] = x[...] * di


def _k0(sdeg, x):
    return _tc(
        _k0_body,
        [_row(1), _row(128)],
        [_row(1), _row(128)],
        (jax.ShapeDtypeStruct((N, 1), jnp.float32),
         jax.ShapeDtypeStruct((N, 128), jnp.float32)),
    )(sdeg, x)


def _dot(a, b):
    return jnp.dot(a, b, preferred_element_type=jnp.float32)


def _k1_body(sg, g0, dinv, W1, b1, W2, g1_o):
    p0 = (sg[...] + g0[...]) * dinv[...]
    h1 = jnp.maximum(_dot(p0, W1[...]) + b1[...], 0.0)
    g1_o[...] = _dot(h1, W2[...]) * dinv[...]


def _k1(sg, g0, dinv, W1, b1, W2):
    return _tc(
        _k1_body,
        [_row(128), _row(128), _row(1),
         _full((128, 256)), _full((1, 256)), _full((256, 128))],
        [_row(128)],
        jax.ShapeDtypeStruct((N, 128), jnp.float32),
    )(sg, g0, dinv, W1, b1, W2)


def _mid_body(sg, g, dinv, b, W, go):
    h = jnp.maximum((sg[...] + g[...]) * dinv[...] + b[...], 0.0)
    go[...] = _dot(h, W[...]) * dinv[...]


def _k2(sg, g1, dinv, b2, W3):
    return _tc(
        _mid_body,
        [_row(128), _row(128), _row(1), _full((1, 128)), _full((128, 64))],
        [_row(64)],
        jax.ShapeDtypeStruct((N, 64), jnp.float32),
    )(sg, g1, dinv, b2, W3)


def _k3(sg, g2, dinv, b3, W4):
    return _tc(
        _mid_body,
        [_row(64), _row(64), _row(1), _full((1, 64)), _full((64, 32))],
        [_row(32)],
        jax.ShapeDtypeStruct((N, 32), jnp.float32),
    )(sg, g2, dinv, b3, W4)


def _k4_body(sg, g, dinv, b, go):
    go[...] = jnp.maximum((sg[...] + g[...]) * dinv[...] + b[...],
                          0.0) * dinv[...]


def _k4(sg, g3, dinv, b4):
    return _tc(
        _k4_body,
        [_row(32), _row(32), _row(1), _full((1, 32))],
        [_row(32)],
        jax.ShapeDtypeStruct((N, 32), jnp.float32),
    )(sg, g3, dinv, b4)


def _k5_body(sg, g, dinv, Wm, bm, Wl, bl, mu_o, ls_o):
    q = (sg[...] + g[...]) * dinv[...]
    mu_o[...] = _dot(q, Wm[...]) + bm[...]
    ls_o[...] = _dot(q, Wl[...]) + bl[...]


def _k5(sg, g4, dinv, Wm, bm, Wl, bl):
    return _tc(
        _k5_body,
        [_row(32), _row(32), _row(1),
         _full((32, 16)), _full((1, 16)), _full((32, 16)), _full((1, 16))],
        [_row(16), _row(16)],
        (jax.ShapeDtypeStruct((N, 16), jnp.float32),
         jax.ShapeDtypeStruct((N, 16), jnp.float32)),
    )(sg, g4, dinv, Wm, bm, Wl, bl)


def kernel(x, edge_index, W1, b1, W2, b2, W3, b3, W4, b4,
           W_mu, b_mu, W_logstd, b_logstd):
    ei = edge_index.astype(jnp.int32)
    pad = EP - E
    srcf = jnp.concatenate([ei[0], jnp.zeros((pad,), jnp.int32)])
    dstf = jnp.concatenate(
        [ei[1], N + (jnp.arange(pad, dtype=jnp.int32) % (ACC - N))])

    stage_s, stage_d, tab = _make_reorder()(srcf, dstf)
    degb = _make_deg_pass()(stage_d, tab)
    sdeg = degb.transpose(1, 0).reshape(ACC, 1)
    dinv, g0 = _k0(sdeg, x)
    g1 = _k1(_prop(128, g0, stage_s, stage_d, tab),
             g0, dinv, W1, b1.reshape(1, -1), W2)
    g2 = _k2(_prop(128, g1, stage_s, stage_d, tab),
             g1, dinv, b2.reshape(1, -1), W3)
    g3 = _k3(_prop(64, g2, stage_s, stage_d, tab),
             g2, dinv, b3.reshape(1, -1), W4)
    g4 = _k4(_prop(32, g3, stage_s, stage_d, tab),
             g3, dinv, b4.reshape(1, -1))
    mu, logstd = _k5(_prop(32, g4, stage_s, stage_d, tab),
                     g4, dinv, W_mu, b_mu.reshape(1, -1),
                     W_logstd, b_logstd.reshape(1, -1))
    return (mu, logstd)


# speedup vs baseline: 1.0975x; 1.0975x over previous
import functools

import jax
import jax.numpy as jnp
from jax import lax
from jax.experimental import pallas as pl
from jax.experimental.pallas import tpu as pltpu
from jax.experimental.pallas import tpu_sc as plsc

N = 10000
E = 320000
NC, NS, L = 2, 16, 16
NW = NC * NS
EROW = 128
CE = 10240
EP = NW * CE
ACC = 10240
LROWS = ACC // NW
TRASH_L = 316
REG = 14336
RB = 1000
GRID = N // RB


def _mesh():
    return plsc.VectorSubcoreMesh(core_axis_name="c", subcore_axis_name="s",
                                  num_cores=NC, num_subcores=NS)


def _sc_params(layout=True):
    return pltpu.CompilerParams(use_tc_tiling_on_sc=False,
                                needs_layout_passes=layout)


def _fill1(buf, n, value, dtype):
    v = jnp.full((L,), value, dtype)
    def body(i, _):
        buf[pl.ds(pl.multiple_of(i * L, L), L)] = v
        return 0
    lax.fori_loop(0, n // L, body, 0)


def _fill2(buf, rows, cols, value, dtype):
    v = jnp.full((L,), value, dtype)
    def body(i, _):
        r = i // (cols // L)
        j = i % (cols // L)
        buf[r, pl.ds(pl.multiple_of(j * L, L), L)] = v
        return 0
    lax.fori_loop(0, rows * (cols // L), body, 0)


def _splat(x):
    return jnp.full((L,), x, jnp.int32)


@functools.lru_cache(maxsize=None)
def _make_reorder():
    @functools.partial(
        pl.kernel,
        out_type=(jax.ShapeDtypeStruct((NW * REG,), jnp.int32),
                  jax.ShapeDtypeStruct((NW * REG,), jnp.int32),
                  jax.ShapeDtypeStruct((NW * 512,), jnp.int32)),
        mesh=_mesh(),
        compiler_params=_sc_params(layout=False),
        scratch_types=[
            pltpu.VMEM((CE,), jnp.int32),
            pltpu.VMEM((CE,), jnp.int32),
            pltpu.VMEM((REG,), jnp.int32),
            pltpu.VMEM((REG,), jnp.int32),
            pltpu.VMEM((512,), jnp.int32),
            pltpu.SMEM((32,), jnp.int32),
        ],
    )
    def reorder(srcf, dstf, stage_s, stage_d, tab, sidx, didx, ss, sd,
                tab_row, cur):
        c = lax.axis_index("c")
        s = lax.axis_index("s")
        w = s * NC + c

        pltpu.sync_copy(srcf.at[pl.ds(pl.multiple_of(w * CE, 8), CE)], sidx)
        pltpu.sync_copy(dstf.at[pl.ds(pl.multiple_of(w * CE, 8), CE)], didx)

        _fill1(ss, REG, 0, jnp.int32)
        _fill1(sd, REG, TRASH_L, jnp.int32)
        _fill1(tab_row, 512, 0, jnp.int32)

        for b in range(32):
            cur[b] = 0
        def count_chunk(t, _):
            dv = didx[pl.ds(pl.multiple_of(t * L, L), L)]
            bv = dv & 31
            for l in range(L):
                b = bv[l]
                cur[b] = cur[b] + 1
            return 0
        lax.fori_loop(0, CE // L, count_chunk, 0)

        off = 0
        for b in range(32):
            cnt = cur[b]
            cntp = ((cnt + EROW - 1) // EROW) * EROW
            plsc.store_scatter(tab_row, [_splat(b * 16)], _splat(off))
            plsc.store_scatter(tab_row, [_splat(b * 16 + 1)], _splat(cntp))
            cur[b] = off
            off = off + cntp

        def place_chunk(t, _):
            to = pl.multiple_of(t * L, L)
            sv = sidx[pl.ds(to, L)]
            dv = didx[pl.ds(to, L)]
            bv = dv & 31
            qv = dv >> 5
            for l in range(L):
                b = bv[l]
                pos = cur[b]
                plsc.store_scatter(ss, [_splat(pos)], _splat(sv[l]))
                plsc.store_scatter(sd, [_splat(pos)], _splat(qv[l]))
                cur[b] = pos + 1
            return 0
        lax.fori_loop(0, CE // L, place_chunk, 0)

        pltpu.sync_copy(ss, stage_s.at[pl.ds(pl.multiple_of(w * REG, 8), REG)])
        pltpu.sync_copy(sd, stage_d.at[pl.ds(pl.multiple_of(w * REG, 8), REG)])
        pltpu.sync_copy(tab_row,
                        tab.at[pl.ds(pl.multiple_of(w * 512, 8), 512)])

    return reorder


@functools.lru_cache(maxsize=None)
def _make_pass(F):
    @functools.partial(
        pl.kernel,
        out_type=jax.ShapeDtypeStruct((NW, LROWS, F), jnp.float32),
        mesh=_mesh(),
        compiler_params=_sc_params(),
        scratch_types=[
            pltpu.VMEM((16,), jnp.int32),
            pltpu.VMEM((EROW,), jnp.int32),
            pltpu.VMEM((EROW,), jnp.int32),
            pltpu.VMEM((EROW, F), jnp.float32),
            pltpu.VMEM((LROWS, F), jnp.float32),
            pltpu.SemaphoreType.DMA,
        ],
    )
    def prop(g_hbm, stage_s, stage_d, tab, out_hbm,
             tab_v, sbuf, dbuf, bufG, acc, sem):
        c = lax.axis_index("c")
        s = lax.axis_index("s")
        B = s * NC + c

        _fill2(acc, LROWS, F, 0.0, jnp.float32)

        def wloop(w, _):
            ent = pl.multiple_of(B * 512 + w * 16, 8)
            pltpu.sync_copy(tab.at[pl.ds(ent, 16)], tab_v)
            tv = tab_v[pl.ds(0, L)]
            off = tv[0]
            nk = tv[1] // EROW
            def kloop(k, _):
                base = pl.multiple_of(w * REG + off + k * EROW, 8)
                pltpu.sync_copy(stage_s.at[pl.ds(base, EROW)], sbuf)
                pltpu.sync_copy(stage_d.at[pl.ds(base, EROW)], dbuf)
                pltpu.async_copy(g_hbm.at[sbuf], bufG, sem).wait()
                def chunk16(t, _):
                    dv = dbuf[pl.ds(pl.multiple_of(t * L, L), L)]
                    for l in range(L):
                        dl = dv[l]
                        e = t * L + l
                        for j in range(F // L):
                            plsc.addupdate(
                                acc.at[dl, pl.ds(j * L, L)],
                                bufG[e, pl.ds(pl.multiple_of(j * L, L), L)])
                    return 0
                lax.fori_loop(0, EROW // L, chunk16, 0)
                return 0
            lax.fori_loop(0, nk, kloop, 0)
            return 0
        lax.fori_loop(0, NW, wloop, 0)

        pltpu.sync_copy(acc, out_hbm.at[B])

    return prop


@functools.lru_cache(maxsize=None)
def _make_deg_pass():
    @functools.partial(
        pl.kernel,
        out_type=jax.ShapeDtypeStruct((NW, LROWS, L), jnp.float32),
        mesh=_mesh(),
        compiler_params=_sc_params(),
        scratch_types=[
            pltpu.VMEM((16,), jnp.int32),
            pltpu.VMEM((EROW,), jnp.int32),
            pltpu.VMEM((LROWS, L), jnp.float32),
        ],
    )
    def degp(stage_d, tab, out_hbm, tab_v, dbuf, acc):
        c = lax.axis_index("c")
        s = lax.axis_index("s")
        B = s * NC + c

        _fill2(acc, LROWS, L, 0.0, jnp.float32)
        one16 = jnp.ones((L,), jnp.float32)

        def wloop(w, _):
            ent = pl.multiple_of(B * 512 + w * 16, 8)
            pltpu.sync_copy(tab.at[pl.ds(ent, 16)], tab_v)
            tv = tab_v[pl.ds(0, L)]
            off = tv[0]
            nk = tv[1] // EROW
            def kloop(k, _):
                base = pl.multiple_of(w * REG + off + k * EROW, 8)
                pltpu.sync_copy(stage_d.at[pl.ds(base, EROW)], dbuf)
                def chunk16(t, _):
                    dv = dbuf[pl.ds(pl.multiple_of(t * L, L), L)]
                    for l in range(L):
                        plsc.addupdate(acc.at[dv[l]], one16)
                    return 0
                lax.fori_loop(0, EROW // L, chunk16, 0)
                return 0
            lax.fori_loop(0, nk, kloop, 0)
            return 0
        lax.fori_loop(0, NW, wloop, 0)

        pltpu.sync_copy(acc, out_hbm.at[B])

    return degp


def _prop(F, g, stage_s, stage_d, tab):
    out = _make_pass(F)(g, stage_s, stage_d, tab)
    return out.transpose(1, 0, 2).reshape(ACC, F)


def _row(F):
    return pl.BlockSpec((RB, F), lambda i: (i, 0))


def _full(shape):
    return pl.BlockSpec(shape, lambda i: tuple(0 for _ in shape))


def _tc(body, in_specs, out_specs, out_shape):
    if not isinstance(out_shape, (tuple, list)):
        out_specs = out_specs[0]
    return pl.pallas_call(body, grid=(GRID,), in_specs=in_specs,
                          out_specs=out_specs, out_shape=out_shape)


def _k0_body(sdeg, x, dinv_o, g0_o):
    deg = sdeg[...] + 1.0
    di = lax.rsqrt(deg)
    dinv_o[...] = di
    g0_o[...] = x[...] * di


def _k0(sdeg, x):
    return _tc(
        _k0_body,
        [_row(1), _row(128)],
        [_row(1), _row(128)],
        (jax.ShapeDtypeStruct((N, 1), jnp.float32),
         jax.ShapeDtypeStruct((N, 128), jnp.float32)),
    )(sdeg, x)


def _dot(a, b):
    return jnp.dot(a, b, preferred_element_type=jnp.float32)


def _k1_body(sg, g0, dinv, W1, b1, W2, g1_o):
    p0 = (sg[...] + g0[...]) * dinv[...]
    h1 = jnp.maximum(_dot(p0, W1[...]) + b1[...], 0.0)
    g1_o[...] = _dot(h1, W2[...]) * dinv[...]


def _k1(sg, g0, dinv, W1, b1, W2):
    return _tc(
        _k1_body,
        [_row(128), _row(128), _row(1),
         _full((128, 256)), _full((1, 256)), _full((256, 128))],
        [_row(128)],
        jax.ShapeDtypeStruct((N, 128), jnp.float32),
    )(sg, g0, dinv, W1, b1, W2)


def _mid_body(sg, g, dinv, b, W, go):
    h = jnp.maximum((sg[...] + g[...]) * dinv[...] + b[...], 0.0)
    go[...] = _dot(h, W[...]) * dinv[...]


def _k2(sg, g1, dinv, b2, W3):
    return _tc(
        _mid_body,
        [_row(128), _row(128), _row(1), _full((1, 128)), _full((128, 64))],
        [_row(64)],
        jax.ShapeDtypeStruct((N, 64), jnp.float32),
    )(sg, g1, dinv, b2, W3)


def _k3(sg, g2, dinv, b3, W4):
    return _tc(
        _mid_body,
        [_row(64), _row(64), _row(1), _full((1, 64)), _full((64, 32))],
        [_row(32)],
        jax.ShapeDtypeStruct((N, 32), jnp.float32),
    )(sg, g2, dinv, b3, W4)


def _k4_body(sg, g, dinv, b, go):
    go[...] = jnp.maximum((sg[...] + g[...]) * dinv[...] + b[...],
                          0.0) * dinv[...]


def _k4(sg, g3, dinv, b4):
    return _tc(
        _k4_body,
        [_row(32), _row(32), _row(1), _full((1, 32))],
        [_row(32)],
        jax.ShapeDtypeStruct((N, 32), jnp.float32),
    )(sg, g3, dinv, b4)


def _k5_body(sg, g, dinv, Wm, bm, Wl, bl, mu_o, ls_o):
    q = (sg[...] + g[...]) * dinv[...]
    mu_o[...] = _dot(q, Wm[...]) + bm[...]
    ls_o[...] = _dot(q, Wl[...]) + bl[...]


def _k5(sg, g4, dinv, Wm, bm, Wl, bl):
    return _tc(
        _k5_body,
        [_row(32), _row(32), _row(1),
         _full((32, 16)), _full((1, 16)), _full((32, 16)), _full((1, 16))],
        [_row(16), _row(16)],
        (jax.ShapeDtypeStruct((N, 16), jnp.float32),
         jax.ShapeDtypeStruct((N, 16), jnp.float32)),
    )(sg, g4, dinv, Wm, bm, Wl, bl)


def kernel(x, edge_index, W1, b1, W2, b2, W3, b3, W4, b4,
           W_mu, b_mu, W_logstd, b_logstd):
    ei = edge_index.astype(jnp.int32)
    pad = EP - E
    srcf = jnp.concatenate([ei[0], jnp.zeros((pad,), jnp.int32)])
    dstf = jnp.concatenate(
        [ei[1], N + (jnp.arange(pad, dtype=jnp.int32) % (ACC - N))])

    stage_s, stage_d, tab = _make_reorder()(srcf, dstf)
    tab = tab.reshape(NW, 32, 16).transpose(1, 0, 2).reshape(NW * 512)
    degb = _make_deg_pass()(stage_d, tab)
    sdeg = degb[:, :, 0].transpose(1, 0).reshape(ACC, 1)
    dinv, g0 = _k0(sdeg[:N], x)
    g1 = _k1(_prop(128, g0, stage_s, stage_d, tab),
             g0, dinv, W1, b1.reshape(1, -1), W2)
    g2 = _k2(_prop(128, g1, stage_s, stage_d, tab),
             g1, dinv, b2.reshape(1, -1), W3)
    g3 = _k3(_prop(64, g2, stage_s, stage_d, tab),
             g2, dinv, b3.reshape(1, -1), W4)
    g4 = _k4(_prop(32, g3, stage_s, stage_d, tab),
             g3, dinv, b4.reshape(1, -1))
    mu, logstd = _k5(_prop(32, g4, stage_s, stage_d, tab),
                     g4, dinv, W_mu, b_mu.reshape(1, -1),
                     W_logstd, b_logstd.reshape(1, -1))
    return (mu, logstd)


# speedup vs baseline: 5.5850x; 5.0887x over previous
import functools

import jax
import jax.numpy as jnp
from jax import lax
from jax.experimental import pallas as pl
from jax.experimental.pallas import tpu as pltpu
from jax.experimental.pallas import tpu_sc as plsc

N = 10000
E = 320000
NC, NS, L = 2, 16, 16
NW = NC * NS
EROW = 128
RW = 80
EP = NW * RW * EROW
NROWS_E = EP // EROW
ACC = 10240
ZC = ACC // NS // EROW
RB = 1000
GRID = N // RB


def _mesh():
    return plsc.VectorSubcoreMesh(core_axis_name="c", subcore_axis_name="s",
                                  num_cores=NC, num_subcores=NS)


def _fill(buf, rows, cols, value):
    v = jnp.full((L,), value, jnp.float32)
    def body(i, _):
        r = i // (cols // L)
        j = i % (cols // L)
        buf[r, pl.ds(j * L, L)] = v
        return 0
    lax.fori_loop(0, rows * (cols // L), body, 0)


@functools.lru_cache(maxsize=None)
def _make_scatter(F):
    @functools.partial(
        pl.kernel,
        out_type=jax.ShapeDtypeStruct((NC, ACC, F), jnp.float32),
        mesh=_mesh(),
        compiler_params=pltpu.CompilerParams(use_tc_tiling_on_sc=False),
        scratch_types=[
            pltpu.VMEM((RW, EROW), jnp.int32),
            pltpu.VMEM((RW, EROW), jnp.int32),
            pltpu.VMEM((EROW, F), jnp.float32),
            pltpu.VMEM((EROW, F), jnp.float32),
            pltpu.VMEM((EROW, F), jnp.float32),
            pltpu.VMEM((EROW, F), jnp.float32),
            pltpu.VMEM_SHARED((ACC, F), jnp.float32),
            pltpu.SemaphoreType.DMA,
            pltpu.SemaphoreType.DMA,
            pltpu.SemaphoreType.DMA,
            pltpu.SemaphoreType.DMA,
            pltpu.SemaphoreType.DMA,
            pltpu.SemaphoreType.DMA,
            pltpu.SemaphoreType.DMA,
            pltpu.SemaphoreType.DMA,
        ],
    )
    def scatter(g_hbm, src_hbm, dst_hbm, out_hbm, sidx, didx, bufA, bufB,
                bufC, bufD, acc, semGA, semGB, semGC, semGD,
                semSA, semSB, semSC, semSD):
        c = lax.axis_index("c")
        s = lax.axis_index("s")
        w = s * NC + c

        _fill(bufA, EROW, F, 0.0)
        def zero_chunk(k, _):
            pltpu.sync_copy(bufA, acc.at[pl.ds(s * (ACC // NS) + k * EROW, EROW)])
            return 0
        lax.fori_loop(0, ZC, zero_chunk, 0)
        plsc.subcore_barrier()

        pltpu.sync_copy(src_hbm.at[pl.ds(w * RW, RW)], sidx)
        pltpu.sync_copy(dst_hbm.at[pl.ds(w * RW, RW)], didx)

        bufs = (bufA, bufB, bufC, bufD)
        gsems = (semGA, semGB, semGC, semGD)
        ssems = (semSA, semSB, semSC, semSD)
        for p in range(4):
            pltpu.async_copy(g_hbm.at[sidx.at[p]], bufs[p], gsems[p])
        def body(t, _):
            for p in range(4):
                r = 4 * t + p
                pltpu.make_async_copy(g_hbm.at[sidx.at[r]], bufs[p],
                                      gsems[p]).wait()
                pltpu.async_copy(bufs[p], acc.at[didx.at[r]], ssems[p],
                                 add=True)
            for p in range(4):
                r = 4 * t + p
                pltpu.make_async_copy(bufs[p], acc.at[didx.at[r]],
                                      ssems[p]).wait()
                pltpu.async_copy(g_hbm.at[sidx.at[r + 4]], bufs[p],
                                 gsems[p])
            return 0
        lax.fori_loop(0, RW // 4 - 1, body, 0)
        last = RW - 4
        for p in range(4):
            pltpu.make_async_copy(g_hbm.at[sidx.at[last + p]], bufs[p],
                                  gsems[p]).wait()
            pltpu.async_copy(bufs[p], acc.at[didx.at[last + p]], ssems[p],
                             add=True)
        for p in range(4):
            pltpu.make_async_copy(bufs[p], acc.at[didx.at[last + p]],
                                  ssems[p]).wait()

        plsc.subcore_barrier()
        def out_chunk(k, _):
            off = s * (ACC // NS) + k * EROW
            pltpu.sync_copy(acc.at[pl.ds(off, EROW)], bufA)
            pltpu.sync_copy(bufA, out_hbm.at[c, pl.ds(off, EROW)])
            return 0
        lax.fori_loop(0, ZC, out_chunk, 0)

    return scatter


DEGF = 16


@functools.lru_cache(maxsize=None)
def _make_deg_scatter():
    @functools.partial(
        pl.kernel,
        out_type=jax.ShapeDtypeStruct((NC, ACC, DEGF), jnp.float32),
        mesh=_mesh(),
        compiler_params=pltpu.CompilerParams(use_tc_tiling_on_sc=False),
        scratch_types=[
            pltpu.VMEM((RW, EROW), jnp.int32),
            pltpu.VMEM((EROW, DEGF), jnp.float32),
            pltpu.VMEM((EROW, DEGF), jnp.float32),
            pltpu.VMEM_SHARED((ACC, DEGF), jnp.float32),
            pltpu.SemaphoreType.DMA,
        ],
    )
    def deg_scatter(dst_hbm, out_hbm, didx, ones, zeros, acc, sem):
        c = lax.axis_index("c")
        s = lax.axis_index("s")
        w = s * NC + c
        _fill(ones, EROW, DEGF, 1.0)
        _fill(zeros, EROW, DEGF, 0.0)
        def zero_chunk(k, _):
            pltpu.sync_copy(zeros,
                            acc.at[pl.ds(s * (ACC // NS) + k * EROW, EROW)])
            return 0
        lax.fori_loop(0, ZC, zero_chunk, 0)
        plsc.subcore_barrier()
        pltpu.sync_copy(dst_hbm.at[pl.ds(w * RW, RW)], didx)
        def body(t, _):
            for p in range(8):
                pltpu.async_copy(ones, acc.at[didx.at[8 * t + p]], sem,
                                 add=True)
            for p in range(8):
                pltpu.make_async_copy(ones, acc.at[didx.at[8 * t + p]],
                                      sem).wait()
            return 0
        lax.fori_loop(0, RW // 8, body, 0)
        plsc.subcore_barrier()
        def out_chunk(k, _):
            off = s * (ACC // NS) + k * EROW
            pltpu.sync_copy(acc.at[pl.ds(off, EROW)], zeros)
            pltpu.sync_copy(zeros, out_hbm.at[c, pl.ds(off, EROW)])
            return 0
        lax.fori_loop(0, ZC, out_chunk, 0)

    return deg_scatter


def _row(F):
    return pl.BlockSpec((RB, F), lambda i: (i, 0))


def _part(F, core):
    return pl.BlockSpec((1, RB, F), lambda i, _c=core: (_c, i, 0))


def _full(shape):
    return pl.BlockSpec(shape, lambda i: tuple(0 for _ in shape))


def _tc(body, in_specs, out_specs, out_shape):
    if not isinstance(out_shape, (tuple, list)):
        out_specs = out_specs[0]
    return pl.pallas_call(body, grid=(GRID,), in_specs=in_specs,
                          out_specs=out_specs, out_shape=out_shape)


def _k0_body(d0, d1, x, dinv_o, g0_o):
    deg = d0[0, :, 0:1] + d1[0, :, 0:1] + 1.0
    di = lax.rsqrt(deg)
    dinv_o[...] = di
    g0_o[...] = x[...] * di


def _k0(degp, x):
    return _tc(
        _k0_body,
        [_part(DEGF, 0), _part(DEGF, 1), _row(128)],
        [_row(1), _row(128)],
        (jax.ShapeDtypeStruct((N, 1), jnp.float32),
         jax.ShapeDtypeStruct((N, 128), jnp.float32)),
    )(degp, degp, x)


def _dot(a, b):
    return jnp.dot(a, b, preferred_element_type=jnp.float32)


def _k1_body(sa0, sa1, sb0, sb1, g0, dinv, W1, b1, W2, g1_o):
    s = jnp.concatenate([sa0[0] + sa1[0], sb0[0] + sb1[0]], axis=-1)
    p0 = (s + g0[...]) * dinv[...]
    h1 = jnp.maximum(_dot(p0, W1[...]) + b1[...], 0.0)
    g1_o[...] = _dot(h1, W2[...]) * dinv[...]


def _k1(sga, sgb, g0, dinv, W1, b1, W2):
    return _tc(
        _k1_body,
        [_part(64, 0), _part(64, 1), _part(64, 0), _part(64, 1),
         _row(128), _row(1),
         _full((128, 256)), _full((1, 256)), _full((256, 128))],
        [_row(128)],
        jax.ShapeDtypeStruct((N, 128), jnp.float32),
    )(sga, sga, sgb, sgb, g0, dinv, W1, b1, W2)


def _k2_body(sa0, sa1, sb0, sb1, g, dinv, b, W, go):
    s = jnp.concatenate([sa0[0] + sa1[0], sb0[0] + sb1[0]], axis=-1)
    h = jnp.maximum((s + g[...]) * dinv[...] + b[...], 0.0)
    go[...] = _dot(h, W[...]) * dinv[...]


def _k2(sga, sgb, g1, dinv, b2, W3):
    return _tc(
        _k2_body,
        [_part(64, 0), _part(64, 1), _part(64, 0), _part(64, 1),
         _row(128), _row(1),
         _full((1, 128)), _full((128, 64))],
        [_row(64)],
        jax.ShapeDtypeStruct((N, 64), jnp.float32),
    )(sga, sga, sgb, sgb, g1, dinv, b2, W3)


def _mid_body(s0, s1, g, dinv, b, W, go):
    h = jnp.maximum((s0[0] + s1[0] + g[...]) * dinv[...] + b[...], 0.0)
    go[...] = _dot(h, W[...]) * dinv[...]


def _k3(sg, g2, dinv, b3, W4):
    return _tc(
        _mid_body,
        [_part(64, 0), _part(64, 1), _row(64), _row(1),
         _full((1, 64)), _full((64, 32))],
        [_row(32)],
        jax.ShapeDtypeStruct((N, 32), jnp.float32),
    )(sg, sg, g2, dinv, b3, W4)


def _k4_body(s0, s1, g, dinv, b, go):
    go[...] = jnp.maximum((s0[0] + s1[0] + g[...]) * dinv[...] + b[...],
                          0.0) * dinv[...]


def _k4(sg, g3, dinv, b4):
    return _tc(
        _k4_body,
        [_part(32, 0), _part(32, 1), _row(32), _row(1), _full((1, 32))],
        [_row(32)],
        jax.ShapeDtypeStruct((N, 32), jnp.float32),
    )(sg, sg, g3, dinv, b4)


def _k5_body(s0, s1, g, dinv, Wm, bm, Wl, bl, mu_o, ls_o):
    q = (s0[0] + s1[0] + g[...]) * dinv[...]
    mu_o[...] = _dot(q, Wm[...]) + bm[...]
    ls_o[...] = _dot(q, Wl[...]) + bl[...]


def _k5(sg, g4, dinv, Wm, bm, Wl, bl):
    return _tc(
        _k5_body,
        [_part(32, 0), _part(32, 1), _row(32), _row(1),
         _full((32, 16)), _full((1, 16)), _full((32, 16)), _full((1, 16))],
        [_row(16), _row(16)],
        (jax.ShapeDtypeStruct((N, 16), jnp.float32),
         jax.ShapeDtypeStruct((N, 16), jnp.float32)),
    )(sg, sg, g4, dinv, Wm, bm, Wl, bl)


def _deg_scatter(dst):
    return _make_deg_scatter()(dst)


def _scatter128(g, src, dst):
    return _make_scatter(128)(g, src, dst)


def _scatter64(g, src, dst):
    return _make_scatter(64)(g, src, dst)


def _scatter32(g, src, dst):
    return _make_scatter(32)(g, src, dst)


def kernel(x, edge_index, W1, b1, W2, b2, W3, b3, W4, b4,
           W_mu, b_mu, W_logstd, b_logstd):
    ei = edge_index.astype(jnp.int32)
    pad = EP - E
    src = jnp.concatenate(
        [ei[0], jnp.zeros((pad,), jnp.int32)]).reshape(NROWS_E, EROW)
    dst = jnp.concatenate(
        [ei[1], N + (jnp.arange(pad, dtype=jnp.int32) % (ACC - N))]
    ).reshape(NROWS_E, EROW)

    degp = _deg_scatter(dst)
    dinv, g0 = _k0(degp, x)
    g1 = _k1(_scatter64(g0[:, :64], src, dst),
             _scatter64(g0[:, 64:], src, dst),
             g0, dinv, W1, b1.reshape(1, -1), W2)
    g2 = _k2(_scatter64(g1[:, :64], src, dst),
             _scatter64(g1[:, 64:], src, dst),
             g1, dinv, b2.reshape(1, -1), W3)
    g3 = _k3(_scatter64(g2, src, dst), g2, dinv, b3.reshape(1, -1), W4)
    g4 = _k4(_scatter32(g3, src, dst), g3, dinv, b4.reshape(1, -1))
    mu, logstd = _k5(_scatter32(g4, src, dst), g4, dinv,
                     W_mu, b_mu.reshape(1, -1), W_logstd,
                     b_logstd.reshape(1, -1))
    return (mu, logstd)


# speedup vs baseline: 5.7186x; 1.0239x over previous
import functools

import jax
import jax.numpy as jnp
from jax import lax
from jax.experimental import pallas as pl
from jax.experimental.pallas import tpu as pltpu
from jax.experimental.pallas import tpu_sc as plsc

N = 10000
E = 320000
NC, NS, L = 2, 16, 16
NW = NC * NS
EROW = 128
RW = 80
EP = NW * RW * EROW
NROWS_E = EP // EROW
ACC = 10240
ZC = ACC // NS // EROW
RB = 1000
GRID = N // RB


def _mesh():
    return plsc.VectorSubcoreMesh(core_axis_name="c", subcore_axis_name="s",
                                  num_cores=NC, num_subcores=NS)


def _fill(buf, rows, cols, value):
    v = jnp.full((L,), value, jnp.float32)
    def body(i, _):
        r = i // (cols // L)
        j = i % (cols // L)
        buf[r, pl.ds(j * L, L)] = v
        return 0
    lax.fori_loop(0, rows * (cols // L), body, 0)


@functools.lru_cache(maxsize=None)
def _make_scatter(F):
    @functools.partial(
        pl.kernel,
        out_type=jax.ShapeDtypeStruct((NC, ACC, F), jnp.float32),
        mesh=_mesh(),
        compiler_params=pltpu.CompilerParams(use_tc_tiling_on_sc=False),
        scratch_types=[
            pltpu.VMEM((RW, EROW), jnp.int32),
            pltpu.VMEM((RW, EROW), jnp.int32),
            pltpu.VMEM((EROW, F), jnp.float32),
            pltpu.VMEM((EROW, F), jnp.float32),
            pltpu.VMEM_SHARED((ACC, F), jnp.float32),
            pltpu.SemaphoreType.DMA,
            pltpu.SemaphoreType.DMA,
        ],
    )
    def scatter(g_hbm, src_hbm, dst_hbm, out_hbm, sidx, didx, bufA, bufB,
                acc, semA, semB):
        c = lax.axis_index("c")
        s = lax.axis_index("s")
        w = s * NC + c

        _fill(bufA, EROW, F, 0.0)
        def zero_chunk(k, _):
            pltpu.sync_copy(bufA, acc.at[pl.ds(s * (ACC // NS) + k * EROW, EROW)])
            return 0
        lax.fori_loop(0, ZC, zero_chunk, 0)
        plsc.subcore_barrier()

        pltpu.sync_copy(src_hbm.at[pl.ds(w * RW, RW)], sidx)
        pltpu.sync_copy(dst_hbm.at[pl.ds(w * RW, RW)], didx)

        pltpu.async_copy(g_hbm.at[sidx.at[0]], bufA, semA)
        def body(t, _):
            i = 2 * t
            pltpu.async_copy(g_hbm.at[sidx.at[i + 1]], bufB, semB)
            pltpu.make_async_copy(g_hbm.at[sidx.at[i]], bufA, semA).wait()
            pltpu.sync_copy(bufA, acc.at[didx.at[i]], add=True)
            pltpu.async_copy(g_hbm.at[sidx.at[i + 2]], bufA, semA)
            pltpu.make_async_copy(g_hbm.at[sidx.at[i + 1]], bufB, semB).wait()
            pltpu.sync_copy(bufB, acc.at[didx.at[i + 1]], add=True)
            return 0
        lax.fori_loop(0, RW // 2 - 1, body, 0)
        pltpu.async_copy(g_hbm.at[sidx.at[RW - 1]], bufB, semB)
        pltpu.make_async_copy(g_hbm.at[sidx.at[RW - 2]], bufA, semA).wait()
        pltpu.sync_copy(bufA, acc.at[didx.at[RW - 2]], add=True)
        pltpu.make_async_copy(g_hbm.at[sidx.at[RW - 1]], bufB, semB).wait()
        pltpu.sync_copy(bufB, acc.at[didx.at[RW - 1]], add=True)

        plsc.subcore_barrier()
        def out_chunk(k, _):
            off = s * (ACC // NS) + k * EROW
            pltpu.sync_copy(acc.at[pl.ds(off, EROW)], bufA)
            pltpu.sync_copy(bufA, out_hbm.at[c, pl.ds(off, EROW)])
            return 0
        lax.fori_loop(0, ZC, out_chunk, 0)

    return scatter


DEGF = 16


@functools.lru_cache(maxsize=None)
def _make_deg_scatter():
    @functools.partial(
        pl.kernel,
        out_type=jax.ShapeDtypeStruct((NC, ACC, DEGF), jnp.float32),
        mesh=_mesh(),
        compiler_params=pltpu.CompilerParams(use_tc_tiling_on_sc=False),
        scratch_types=[
            pltpu.VMEM((RW, EROW), jnp.int32),
            pltpu.VMEM((EROW, DEGF), jnp.float32),
            pltpu.VMEM((EROW, DEGF), jnp.float32),
            pltpu.VMEM_SHARED((ACC, DEGF), jnp.float32),
        ],
    )
    def deg_scatter(dst_hbm, out_hbm, didx, ones, zeros, acc):
        c = lax.axis_index("c")
        s = lax.axis_index("s")
        w = s * NC + c
        _fill(ones, EROW, DEGF, 1.0)
        _fill(zeros, EROW, DEGF, 0.0)
        def zero_chunk(k, _):
            pltpu.sync_copy(zeros,
                            acc.at[pl.ds(s * (ACC // NS) + k * EROW, EROW)])
            return 0
        lax.fori_loop(0, ZC, zero_chunk, 0)
        plsc.subcore_barrier()
        pltpu.sync_copy(dst_hbm.at[pl.ds(w * RW, RW)], didx)
        def body(i, _):
            pltpu.sync_copy(ones, acc.at[didx.at[i]], add=True)
            return 0
        lax.fori_loop(0, RW, body, 0)
        plsc.subcore_barrier()
        def out_chunk(k, _):
            off = s * (ACC // NS) + k * EROW
            pltpu.sync_copy(acc.at[pl.ds(off, EROW)], zeros)
            pltpu.sync_copy(zeros, out_hbm.at[c, pl.ds(off, EROW)])
            return 0
        lax.fori_loop(0, ZC, out_chunk, 0)

    return deg_scatter


def _row(F):
    return pl.BlockSpec((RB, F), lambda i: (i, 0))


def _part(F, core):
    return pl.BlockSpec((1, RB, F), lambda i, _c=core: (_c, i, 0))


def _full(shape):
    return pl.BlockSpec(shape, lambda i: tuple(0 for _ in shape))


def _tc(body, in_specs, out_specs, out_shape):
    if not isinstance(out_shape, (tuple, list)):
        out_specs = out_specs[0]
    return pl.pallas_call(body, grid=(GRID,), in_specs=in_specs,
                          out_specs=out_specs, out_shape=out_shape)


def _k0_body(d0, d1, x, dinv_o, g0_o):
    deg = d0[0, :, 0:1] + d1[0, :, 0:1] + 1.0
    di = lax.rsqrt(deg)
    dinv_o[...] = di
    g0_o[...] = x[...] * di


def _k0(degp, x):
    return _tc(
        _k0_body,
        [_part(DEGF, 0), _part(DEGF, 1), _row(128)],
        [_row(1), _row(128)],
        (jax.ShapeDtypeStruct((N, 1), jnp.float32),
         jax.ShapeDtypeStruct((N, 128), jnp.float32)),
    )(degp, degp, x)


def _dot(a, b):
    return jnp.dot(a, b, preferred_element_type=jnp.float32)


def _k1_body(sa0, sa1, sb0, sb1, g0, dinv, W1, b1, W2, g1_o):
    s = jnp.concatenate([sa0[0] + sa1[0], sb0[0] + sb1[0]], axis=-1)
    p0 = (s + g0[...]) * dinv[...]
    h1 = jnp.maximum(_dot(p0, W1[...]) + b1[...], 0.0)
    g1_o[...] = _dot(h1, W2[...]) * dinv[...]


def _k1(sga, sgb, g0, dinv, W1, b1, W2):
    return _tc(
        _k1_body,
        [_part(64, 0), _part(64, 1), _part(64, 0), _part(64, 1),
         _row(128), _row(1),
         _full((128, 256)), _full((1, 256)), _full((256, 128))],
        [_row(128)],
        jax.ShapeDtypeStruct((N, 128), jnp.float32),
    )(sga, sga, sgb, sgb, g0, dinv, W1, b1, W2)


def _k2_body(sa0, sa1, sb0, sb1, g, dinv, b, W, go):
    s = jnp.concatenate([sa0[0] + sa1[0], sb0[0] + sb1[0]], axis=-1)
    h = jnp.maximum((s + g[...]) * dinv[...] + b[...], 0.0)
    go[...] = _dot(h, W[...]) * dinv[...]


def _k2(sga, sgb, g1, dinv, b2, W3):
    return _tc(
        _k2_body,
        [_part(64, 0), _part(64, 1), _part(64, 0), _part(64, 1),
         _row(128), _row(1),
         _full((1, 128)), _full((128, 64))],
        [_row(64)],
        jax.ShapeDtypeStruct((N, 64), jnp.float32),
    )(sga, sga, sgb, sgb, g1, dinv, b2, W3)


def _mid_body(s0, s1, g, dinv, b, W, go):
    h = jnp.maximum((s0[0] + s1[0] + g[...]) * dinv[...] + b[...], 0.0)
    go[...] = _dot(h, W[...]) * dinv[...]


def _k3(sg, g2, dinv, b3, W4):
    return _tc(
        _mid_body,
        [_part(64, 0), _part(64, 1), _row(64), _row(1),
         _full((1, 64)), _full((64, 32))],
        [_row(32)],
        jax.ShapeDtypeStruct((N, 32), jnp.float32),
    )(sg, sg, g2, dinv, b3, W4)


def _k4_body(s0, s1, g, dinv, b, go):
    go[...] = jnp.maximum((s0[0] + s1[0] + g[...]) * dinv[...] + b[...],
                          0.0) * dinv[...]


def _k4(sg, g3, dinv, b4):
    return _tc(
        _k4_body,
        [_part(32, 0), _part(32, 1), _row(32), _row(1), _full((1, 32))],
        [_row(32)],
        jax.ShapeDtypeStruct((N, 32), jnp.float32),
    )(sg, sg, g3, dinv, b4)


def _k5_body(s0, s1, g, dinv, Wm, bm, Wl, bl, mu_o, ls_o):
    q = (s0[0] + s1[0] + g[...]) * dinv[...]
    mu_o[...] = _dot(q, Wm[...]) + bm[...]
    ls_o[...] = _dot(q, Wl[...]) + bl[...]


def _k5(sg, g4, dinv, Wm, bm, Wl, bl):
    return _tc(
        _k5_body,
        [_part(32, 0), _part(32, 1), _row(32), _row(1),
         _full((32, 16)), _full((1, 16)), _full((32, 16)), _full((1, 16))],
        [_row(16), _row(16)],
        (jax.ShapeDtypeStruct((N, 16), jnp.float32),
         jax.ShapeDtypeStruct((N, 16), jnp.float32)),
    )(sg, sg, g4, dinv, Wm, bm, Wl, bl)


def _deg_scatter(dst):
    return _make_deg_scatter()(dst)


def _scatter128(g, src, dst):
    return _make_scatter(128)(g, src, dst)


def _scatter64(g, src, dst):
    return _make_scatter(64)(g, src, dst)


def _scatter32(g, src, dst):
    return _make_scatter(32)(g, src, dst)


def kernel(x, edge_index, W1, b1, W2, b2, W3, b3, W4, b4,
           W_mu, b_mu, W_logstd, b_logstd):
    ei = edge_index.astype(jnp.int32)
    pad = EP - E
    src = jnp.concatenate(
        [ei[0], jnp.zeros((pad,), jnp.int32)]).reshape(NROWS_E, EROW)
    dst = jnp.concatenate(
        [ei[1], N + (jnp.arange(pad, dtype=jnp.int32) % (ACC - N))]
    ).reshape(NROWS_E, EROW)

    degp = _deg_scatter(dst)
    dinv, g0 = _k0(degp, x)
    g1 = _k1(_scatter64(g0[:, :64], src, dst),
             _scatter64(g0[:, 64:], src, dst),
             g0, dinv, W1, b1.reshape(1, -1), W2)
    g2 = _k2(_scatter64(g1[:, :64], src, dst),
             _scatter64(g1[:, 64:], src, dst),
             g1, dinv, b2.reshape(1, -1), W3)
    g3 = _k3(_scatter64(g2, src, dst), g2, dinv, b3.reshape(1, -1), W4)
    g4 = _k4(_scatter32(g3, src, dst), g3, dinv, b4.reshape(1, -1))
    mu, logstd = _k5(_scatter32(g4, src, dst), g4, dinv,
                     W_mu, b_mu.reshape(1, -1), W_logstd,
                     b_logstd.reshape(1, -1))
    return (mu, logstd)
